# baseline (speedup 1.0000x reference)
import jax
import jax.numpy as jnp
from jax.experimental import pallas as pl
from jax.experimental.pallas import tpu as pltpu

_N = 4096
_C = 1024
_F = 4096
_MBM = 16
_BM = 128
_MB = _N // _BM
_R = _BM // _MBM
_NMB = _N // _MBM
_K = 1024
_TB = 256
_NT = _N // _TB
_W1C = _F // 8
_W2C = _C // 8

_INTERPRET = False


def _fused_kernel(x_ref, b1_ref, w1_ref, w2_ref, bmc_ref, pa_ref, oc_ref,
                  out_ref, bm_ref, mdiff_ref, thr_ref, w1b_ref, w2b_ref):
    i = pl.program_id(0)

    @pl.when(i < 8)
    def _pack_w1():
        w1b_ref[pl.ds(_W1C * i, _W1C), :] = w1_ref[...].astype(jnp.bfloat16)

    @pl.when(jnp.logical_and(i >= 8, i < 16))
    def _pack_w2():
        w2b_ref[pl.ds(_W2C * (i - 8), _W2C), :] = (
            w2_ref[...].astype(jnp.bfloat16))

    @pl.when(i < _NT)
    def _phase1():
        bm_ref[pl.ds(_MBM * i, _MBM), :] = (
            x_ref[...].reshape(_MBM, _MBM, _C).mean(axis=1))

    @pl.when(i == _NT)
    def _select():
        t = jax.lax.dot_general(bm_ref[...].astype(jnp.bfloat16), w1b_ref[...],
                                (((1,), (1,)), ((), ())),
                                preferred_element_type=jnp.float32)
        t = t + b1_ref[...]
        md = jnp.abs(t - bmc_ref[...])
        mdiff_ref[...] = md.reshape(_MB, _R, _F).sum(axis=1)
        bits = jax.lax.bitcast_convert_type(mdiff_ref[...], jnp.int32)

        def body(_, carry):
            lo, hi = carry
            mid = lo + (hi - lo) // 2
            cnt = jnp.sum((bits >= mid).astype(jnp.int32), axis=1,
                          keepdims=True)
            ge = cnt >= _K
            return jnp.where(ge, mid, lo), jnp.where(ge, hi, mid)

        lo0 = jnp.zeros((_MB, 1), jnp.int32)
        hi0 = jnp.full((_MB, 1), 0x7F800000, jnp.int32)
        lo, _hi = jax.lax.fori_loop(0, 31, body, (lo0, hi0))
        thr_ref[...] = jax.lax.bitcast_convert_type(lo, jnp.float32)

    @pl.when(i > _NT)
    def _phase2():
        m = i - (_NT + 1)
        xv = x_ref[...].astype(jnp.bfloat16)
        mid = jax.lax.dot_general(xv, w1b_ref[...],
                                  (((1,), (1,)), ((), ())),
                                  preferred_element_type=jnp.float32)
        mid = mid + b1_ref[...]
        act = jax.nn.gelu(mid)
        m0 = (mdiff_ref[pl.ds(2 * m, 1), :]
              >= thr_ref[pl.ds(2 * m, 1), :]).astype(jnp.float32)
        m1 = (mdiff_ref[pl.ds(2 * m + 1, 1), :]
              >= thr_ref[pl.ds(2 * m + 1, 1), :]).astype(jnp.float32)
        condf = (jax.lax.broadcasted_iota(jnp.int32, (_TB, 1), 0)
                 < _BM).astype(jnp.float32)
        mask = m0 * condf + m1 * (1.0 - condf)
        delta = (act - pa_ref[...]) * mask
        part = jax.lax.dot_general(delta.astype(jnp.bfloat16), w2b_ref[...],
                                   (((1,), (1,)), ((), ())),
                                   preferred_element_type=jnp.float32)
        out_ref[...] = oc_ref[...] + part


def kernel(x, W1, b1, W2, b2, blockmean_mid_cache, pa_cache, out_cache):
    x2 = x.reshape(_N, _C)
    bmc = blockmean_mid_cache.reshape(_NMB, _F)
    b1r = b1.reshape(1, _F)
    pa2 = pa_cache.reshape(_N, _F)
    oc2 = out_cache.reshape(_N, _C)

    def _xmap(i):
        return (jnp.where(i < _NT, i, jnp.maximum(i - (_NT + 1), 0)), 0)

    def _p2map(i):
        return (jnp.maximum(i - (_NT + 1), 0), 0)

    out = pl.pallas_call(
        _fused_kernel,
        grid=(2 * _NT + 1,),
        in_specs=[
            pl.BlockSpec((_TB, _C), _xmap),
            pl.BlockSpec((1, _F), lambda i: (0, 0)),
            pl.BlockSpec((_W1C, _C), lambda i: (jnp.minimum(i, 7), 0)),
            pl.BlockSpec((_W2C, _F),
                         lambda i: (jnp.clip(i - 8, 0, 7), 0)),
            pl.BlockSpec((_NMB, _F), lambda i: (0, 0)),
            pl.BlockSpec((_TB, _F), _p2map),
            pl.BlockSpec((_TB, _C), _p2map),
        ],
        out_specs=pl.BlockSpec((_TB, _C), _p2map),
        out_shape=jax.ShapeDtypeStruct((_N, _C), jnp.float32),
        scratch_shapes=[
            pltpu.VMEM((_NMB, _C), jnp.float32),
            pltpu.VMEM((_MB, _F), jnp.float32),
            pltpu.VMEM((_MB, 1), jnp.float32),
            pltpu.VMEM((_F, _C), jnp.bfloat16),
            pltpu.VMEM((_C, _F), jnp.bfloat16),
        ],
        compiler_params=pltpu.CompilerParams(
            dimension_semantics=("arbitrary",),
            vmem_limit_bytes=100 * 1024 * 1024),
        interpret=_INTERPRET,
    )(x2, b1r, W1, W2, bmc, pa2, oc2)

    return out.reshape(1, _N, _C)
